# baseline (speedup 1.0000x reference)
import functools

import jax
import jax.numpy as jnp
from jax import lax
from jax.experimental import pallas as pl
from jax.experimental.pallas import tpu as pltpu
from jax.experimental.pallas import tpu_sc as plsc

DIM = 64
PROMPT_LEN = 20
BATCH = 4096
D = PROMPT_LEN * DIM

_NC = 2
_NS = 16
_NW = _NC * _NS
_K = 4
_CB = BATCH // _K
_BPW = _CB // _NW
_C = 32
_NCHUNK = _BPW // _C
_NB = min(3, max(_NCHUNK, 1))


def _make_gather():
    mesh = plsc.VectorSubcoreMesh(
        core_axis_name="c", subcore_axis_name="s", num_cores=_NC)

    @functools.partial(
        pl.kernel,
        mesh=mesh,
        out_type=jax.ShapeDtypeStruct((_CB, D), jnp.float32),
        scratch_types=[
            pltpu.VMEM((_NCHUNK, _C), jnp.int32),
        ]
        + [pltpu.VMEM((_C, D), jnp.float32) for _ in range(_NB)]
        + [pltpu.SemaphoreType.DMA for _ in range(2 * _NB)],
    )
    def gather_kernel(idx_hbm, table_hbm, out_hbm, idx_v, *rest):
        bufs = rest[:_NB]
        gsems = rest[_NB:2 * _NB]
        osems = rest[2 * _NB:]
        wid = lax.axis_index("s") * _NC + lax.axis_index("c")
        base = wid * _BPW
        pltpu.sync_copy(idx_hbm.at[wid], idx_v)

        def gather(c):
            b = c % _NB
            return pltpu.async_copy(table_hbm.at[idx_v.at[c]], bufs[b], gsems[b])

        def store(c):
            b = c % _NB
            return pltpu.async_copy(
                bufs[b], out_hbm.at[pl.ds(base + c * _C, _C)], osems[b])

        g_pend = [None] * _NB
        s_pend = [None] * _NB
        for c in range(min(_NB, _NCHUNK)):
            g_pend[c % _NB] = gather(c)
        for c in range(_NCHUNK):
            b = c % _NB
            g_pend[b].wait()
            g_pend[b] = None
            s_pend[b] = store(c)
            if c + _NB < _NCHUNK:
                s_pend[b].wait()
                s_pend[b] = None
                g_pend[b] = gather(c + _NB)
        for h in s_pend:
            if h is not None:
                h.wait()

    return gather_kernel


_gather = _make_gather()


def kernel(row_idx, emb_weight):
    idx = row_idx.astype(jnp.int32).reshape(_K, _NW, _NCHUNK, _C)
    parts = [_gather(idx[k], emb_weight) for k in range(_K)]
    out = jnp.concatenate(parts, axis=0)
    return out.reshape(BATCH, PROMPT_LEN, DIM)


# speedup vs baseline: 1.4812x; 1.4812x over previous
import functools

import jax
import jax.numpy as jnp
from jax import lax
from jax.experimental import pallas as pl
from jax.experimental.pallas import tpu as pltpu
from jax.experimental.pallas import tpu_sc as plsc

DIM = 64
PROMPT_LEN = 20
BATCH = 4096
D = PROMPT_LEN * DIM

_NC = 2
_NS = 16
_NW = _NC * _NS
_BPW = BATCH // _NW
_C = 32
_NCHUNK = _BPW // _C
_NB = 3


def _make_gather():
    mesh = plsc.VectorSubcoreMesh(
        core_axis_name="c", subcore_axis_name="s", num_cores=_NC)

    @functools.partial(
        pl.kernel,
        mesh=mesh,
        out_type=jax.ShapeDtypeStruct((BATCH, D), jnp.float32),
        scratch_types=[
            pltpu.VMEM((_BPW,), jnp.int32),
        ]
        + [pltpu.VMEM((_C, D), jnp.float32) for _ in range(_NB)]
        + [pltpu.SemaphoreType.DMA for _ in range(2 * _NB)],
    )
    def gather_kernel(idx_hbm, table_hbm, out_hbm, idx_v, *rest):
        bufs = rest[:_NB]
        gsems = rest[_NB:2 * _NB]
        osems = rest[2 * _NB:]
        wid = lax.axis_index("s") * _NC + lax.axis_index("c")
        base = wid * _BPW
        pltpu.sync_copy(idx_hbm.at[pl.ds(base, _BPW)], idx_v)

        def gather(c):
            b = c % _NB
            return pltpu.async_copy(
                table_hbm.at[idx_v.at[pl.ds(c * _C, _C)]], bufs[b], gsems[b])

        def store(c):
            b = c % _NB
            return pltpu.async_copy(
                bufs[b], out_hbm.at[pl.ds(base + c * _C, _C)], osems[b])

        g_pend = [None] * _NB
        s_pend = [None] * _NB
        for c in range(min(_NB, _NCHUNK)):
            g_pend[c % _NB] = gather(c)
        for c in range(_NCHUNK):
            b = c % _NB
            g_pend[b].wait()
            g_pend[b] = None
            s_pend[b] = store(c)
            if c + _NB < _NCHUNK:
                s_pend[b].wait()
                s_pend[b] = None
                g_pend[b] = gather(c + _NB)
        for h in s_pend:
            if h is not None:
                h.wait()

    return gather_kernel


_gather = _make_gather()


def kernel(row_idx, emb_weight):
    out = _gather(row_idx.astype(jnp.int32), emb_weight)
    return out.reshape(BATCH, PROMPT_LEN, DIM)
